# Initial kernel scaffold; baseline (speedup 1.0000x reference)
import jax
import jax.numpy as jnp
from jax.experimental import pallas as pl


def kernel(x, Wr, W1, W2):
    raise NotImplementedError("write your pallas kernel here")


# speedup vs baseline: 2.1373x; 2.1373x over previous
import functools

import jax
import jax.numpy as jnp
from jax import lax
from jax.experimental import pallas as pl
from jax.experimental.pallas import tpu as pltpu
from jax.experimental.pallas import tpu_sc as plsc

B, S, D = 2, 2048, 1024
FF = 4096
E = 8
T = B * S
CAP = T // E
TB = 1024
NTB = T // TB
FFB = 1024
NFFB = FF // FFB

NC, NS = 2, 16
NW = NC * NS
RPT = T // NW
GCH = 64


def _router_body(x_ref, wr_ref, slot_ref, prob_ref, carry_ref):
    pid = pl.program_id(0)

    @pl.when(pid == 0)
    def _():
        carry_ref[...] = jnp.zeros_like(carry_ref)

    xb = x_ref[...]
    logits = lax.dot_general(xb, wr_ref[...],
                             (((1,), (1,)), ((), ())),
                             preferred_element_type=jnp.float32)
    lmax = jnp.max(logits, axis=1, keepdims=True)
    ssum = jnp.sum(jnp.exp(logits - lmax), axis=1)
    iota_e = lax.broadcasted_iota(jnp.int32, (TB, E), 1)
    is_max = logits == lmax
    idx = jnp.min(jnp.where(is_max, iota_e, E), axis=1)
    onehot = (iota_e == idx[:, None])

    r_io = lax.broadcasted_iota(jnp.int32, (TB, TB), 0)
    c_io = lax.broadcasted_iota(jnp.int32, (TB, TB), 1)
    ltri = (r_io >= c_io).astype(jnp.bfloat16)
    csum = lax.dot_general(ltri, onehot.astype(jnp.bfloat16),
                           (((1,), (0,)), ((), ())),
                           preferred_element_type=jnp.float32)
    rank_in_blk = jnp.sum(csum * onehot.astype(jnp.float32), axis=1) - 1.0
    carry = carry_ref[...]
    base = jnp.sum(carry * onehot.astype(jnp.float32), axis=1)
    rank = (rank_in_blk + base).astype(jnp.int32)
    carry_ref[...] = carry + jnp.sum(onehot.astype(jnp.float32), axis=0,
                                     keepdims=True)

    slot = jnp.where(rank < CAP, idx * CAP + rank, E * CAP)
    slot_ref[...] = slot.reshape(1, 1, TB)
    prob_ref[...] = (1.0 / ssum).reshape(1, 1, TB)


def _router(x_flat, Wr):
    slot, prob = pl.pallas_call(
        _router_body,
        grid=(NTB,),
        in_specs=[
            pl.BlockSpec((TB, D), lambda i: (i, 0)),
            pl.BlockSpec((E, D), lambda i: (0, 0)),
        ],
        out_specs=[
            pl.BlockSpec((1, 1, TB), lambda i: (i, 0, 0)),
            pl.BlockSpec((1, 1, TB), lambda i: (i, 0, 0)),
        ],
        out_shape=[
            jax.ShapeDtypeStruct((NTB, 1, TB), jnp.int32),
            jax.ShapeDtypeStruct((NTB, 1, TB), jnp.float32),
        ],
        scratch_shapes=[pltpu.VMEM((1, E), jnp.float32)],
    )(x_flat, Wr)
    return slot.reshape(T), prob.reshape(T)


def _dispatch_body(x_hbm, slot_hbm, prob_hbm, xe_hbm, pslot_hbm,
                   slot_v, prob_v, ids_v, ps_v, rows_v, sem):
    wid = lax.axis_index("s") * NC + lax.axis_index("c")
    base = wid * RPT

    pltpu.sync_copy(slot_hbm, slot_v)
    pltpu.sync_copy(prob_hbm, prob_v)

    zero16 = jnp.zeros((16,), jnp.int32)

    def init_body(c, _):
        ids_v[pl.ds(c * 16, 16)] = zero16
        return 0

    lax.fori_loop(0, T // 16, init_body, 0)

    i16 = lax.iota(jnp.int32, 16)

    def scat_body(c, _):
        sv = slot_v[pl.ds(c * 16, 16)]
        tok = i16 + c * 16
        m = sv < T
        plsc.store_scatter(ids_v, [sv], tok, mask=m)
        plsc.store_scatter(ps_v, [sv], prob_v[pl.ds(c * 16, 16)], mask=m)
        return 0

    lax.fori_loop(0, T // 16, scat_body, 0)

    pltpu.sync_copy(ps_v.at[pl.ds(base, RPT)], pslot_hbm.at[pl.ds(base, RPT)])

    def gat_body(c, _):
        idx = ids_v.at[pl.ds(base + c * GCH, GCH)]
        pltpu.async_copy(x_hbm.at[idx], rows_v, sem).wait()
        pltpu.sync_copy(rows_v, xe_hbm.at[pl.ds(base + c * GCH, GCH), :])
        return 0

    lax.fori_loop(0, RPT // GCH, gat_body, 0)


def _dispatch(x_flat, slot, prob):
    mesh = plsc.VectorSubcoreMesh(core_axis_name="c", subcore_axis_name="s")
    k = pl.kernel(
        _dispatch_body,
        compiler_params=pltpu.CompilerParams(needs_layout_passes=False),
        out_type=[
            jax.ShapeDtypeStruct((T, D), jnp.float32),
            jax.ShapeDtypeStruct((T,), jnp.float32),
        ],
        mesh=mesh,
        scratch_types=[
            pltpu.VMEM((T,), jnp.int32),
            pltpu.VMEM((T,), jnp.float32),
            pltpu.VMEM((T,), jnp.int32),
            pltpu.VMEM((T,), jnp.float32),
            pltpu.VMEM((GCH, D), jnp.float32),
            pltpu.SemaphoreType.DMA,
        ],
    )
    return k(x_flat, slot, prob)


def _ffn_body(x_ref, w1_ref, w2_ref, p_ref, out_ref, acc_ref):
    e = pl.program_id(0)
    f = pl.program_id(1)

    @pl.when(e < E)
    def _():
        xb = x_ref[0].astype(jnp.bfloat16)
        w1 = w1_ref[0].astype(jnp.bfloat16)
        h = lax.dot_general(xb, w1, (((1,), (1,)), ((), ())),
                            preferred_element_type=jnp.float32)
        h = jnp.maximum(h, 0.0).astype(jnp.bfloat16)
        w2 = w2_ref[0].astype(jnp.bfloat16)
        part = lax.dot_general(h, w2, (((1,), (1,)), ((), ())),
                               preferred_element_type=jnp.float32)

        @pl.when(f == 0)
        def _():
            acc_ref[...] = part

        @pl.when(f > 0)
        def _():
            acc_ref[...] = acc_ref[...] + part

        @pl.when(f == NFFB - 1)
        def _():
            pv = p_ref[0, 0][:, None]
            out_ref[0] = acc_ref[...] * pv

    @pl.when(e == E)
    def _():
        out_ref[0] = jnp.zeros_like(out_ref[0])


def _ffn(xe, W1, W2, pslot):
    xe3 = xe.reshape(E, CAP, D)
    p3 = pslot.reshape(E, 1, CAP)
    clamp = lambda e: jnp.minimum(e, E - 1)
    out = pl.pallas_call(
        _ffn_body,
        grid=(E + 1, NFFB),
        in_specs=[
            pl.BlockSpec((1, CAP, D), lambda e, f: (clamp(e), 0, 0)),
            pl.BlockSpec((1, FFB, D), lambda e, f: (clamp(e), f, 0)),
            pl.BlockSpec((1, D, FFB), lambda e, f: (clamp(e), 0, f)),
            pl.BlockSpec((1, 1, CAP), lambda e, f: (clamp(e), 0, 0)),
        ],
        out_specs=pl.BlockSpec((1, CAP, D), lambda e, f: (e, 0, 0)),
        out_shape=jax.ShapeDtypeStruct((E + 1, CAP, D), jnp.float32),
        scratch_shapes=[pltpu.VMEM((CAP, D), jnp.float32)],
    )(xe3, W1, W2, p3)
    return out.reshape((E + 1) * CAP, D)


def _collect_body(res_hbm, slot_hbm, y_hbm, slot_v, rows_v, sem):
    wid = lax.axis_index("s") * NC + lax.axis_index("c")
    base = wid * RPT

    pltpu.sync_copy(slot_hbm.at[pl.ds(base, RPT)], slot_v)

    def gat_body(c, _):
        idx = slot_v.at[pl.ds(c * GCH, GCH)]
        pltpu.async_copy(res_hbm.at[idx], rows_v, sem).wait()
        pltpu.sync_copy(rows_v, y_hbm.at[pl.ds(base + c * GCH, GCH), :])
        return 0

    lax.fori_loop(0, RPT // GCH, gat_body, 0)


def _collect(result, slot):
    mesh = plsc.VectorSubcoreMesh(core_axis_name="c", subcore_axis_name="s")
    k = pl.kernel(
        _collect_body,
        out_type=jax.ShapeDtypeStruct((T, D), jnp.float32),
        mesh=mesh,
        scratch_types=[
            pltpu.VMEM((RPT,), jnp.int32),
            pltpu.VMEM((GCH, D), jnp.float32),
            pltpu.SemaphoreType.DMA,
        ],
    )
    return k(result, slot)


@jax.jit
def kernel(x, Wr, W1, W2):
    x_flat = x.reshape(T, D)
    slot, prob = _router(x_flat, Wr)
    xe, pslot = _dispatch(x_flat, slot, prob)
    result = _ffn(xe, W1, W2, pslot)
    y = _collect(result, slot)
    return y.reshape(B, S, D)


# speedup vs baseline: 2.3086x; 1.0802x over previous
import functools

import jax
import jax.numpy as jnp
from jax import lax
from jax.experimental import pallas as pl
from jax.experimental.pallas import tpu as pltpu
from jax.experimental.pallas import tpu_sc as plsc

B, S, D = 2, 2048, 1024
FF = 4096
E = 8
T = B * S
CAP = T // E
TB = 1024
NTB = T // TB
FFB = 1024
NFFB = FF // FFB

NC, NS = 2, 16
NW = NC * NS
RPT = T // NW
GCH = 32
NCH = RPT // GCH


def _pipelined_gather(src_hbm, idx_ref, dst_hbm, dst_base, rows_v,
                      gsems, osems):
    gets = [None] * NCH
    puts = [None] * NCH
    for c in range(NCH):
        if c >= 2:
            puts[c - 2].wait()
        gets[c] = pltpu.async_copy(
            src_hbm.at[idx_ref.at[pl.ds(c * GCH, GCH)]],
            rows_v.at[c % 2], gsems[c % 2])
        if c >= 1:
            gets[c - 1].wait()
            puts[c - 1] = pltpu.async_copy(
                rows_v.at[(c - 1) % 2],
                dst_hbm.at[pl.ds(dst_base + (c - 1) * GCH, GCH), :],
                osems[(c - 1) % 2])
    gets[NCH - 1].wait()
    puts[NCH - 1] = pltpu.async_copy(
        rows_v.at[(NCH - 1) % 2],
        dst_hbm.at[pl.ds(dst_base + (NCH - 1) * GCH, GCH), :],
        osems[(NCH - 1) % 2])
    if NCH >= 2:
        puts[NCH - 2].wait()
    puts[NCH - 1].wait()


def _router_body(x_ref, wr_ref, slot_ref, prob_ref, carry_ref):
    pid = pl.program_id(0)

    @pl.when(pid == 0)
    def _():
        carry_ref[...] = jnp.zeros_like(carry_ref)

    xb = x_ref[...]
    logits = lax.dot_general(wr_ref[...], xb,
                             (((1,), (1,)), ((), ())),
                             preferred_element_type=jnp.float32)
    lmax = jnp.max(logits, axis=0, keepdims=True)
    ssum = jnp.sum(jnp.exp(logits - lmax), axis=0)
    iota_e = lax.broadcasted_iota(jnp.int32, (E, TB), 0)
    is_max = logits == lmax
    idx = jnp.min(jnp.where(is_max, iota_e, E), axis=0)
    onehot = (iota_e == idx[None, :])

    r_io = lax.broadcasted_iota(jnp.int32, (TB, TB), 0)
    c_io = lax.broadcasted_iota(jnp.int32, (TB, TB), 1)
    utri = (r_io <= c_io).astype(jnp.bfloat16)
    csum = lax.dot_general(onehot.astype(jnp.bfloat16), utri,
                           (((1,), (0,)), ((), ())),
                           preferred_element_type=jnp.float32)
    ohf = onehot.astype(jnp.float32)
    rank_in_blk = jnp.sum(csum * ohf, axis=0) - 1.0
    carry = carry_ref[...]
    base = jnp.sum(carry * ohf, axis=0)
    rank = (rank_in_blk + base).astype(jnp.int32)
    carry_ref[...] = carry + jnp.sum(ohf, axis=1, keepdims=True)

    slot = jnp.where(rank < CAP, idx * CAP + rank, E * CAP)
    slot_ref[...] = slot.reshape(1, 1, TB)
    prob_ref[...] = (1.0 / ssum).reshape(1, 1, TB)


def _router(x_flat, Wr):
    slot, prob = pl.pallas_call(
        _router_body,
        grid=(NTB,),
        in_specs=[
            pl.BlockSpec((TB, D), lambda i: (i, 0)),
            pl.BlockSpec((E, D), lambda i: (0, 0)),
        ],
        out_specs=[
            pl.BlockSpec((1, 1, TB), lambda i: (i, 0, 0)),
            pl.BlockSpec((1, 1, TB), lambda i: (i, 0, 0)),
        ],
        out_shape=[
            jax.ShapeDtypeStruct((NTB, 1, TB), jnp.int32),
            jax.ShapeDtypeStruct((NTB, 1, TB), jnp.float32),
        ],
        scratch_shapes=[pltpu.VMEM((E, 1), jnp.float32)],
    )(x_flat, Wr)
    return slot.reshape(T), prob.reshape(T)


def _dispatch_body(x_hbm, slot_hbm, prob_hbm, xe_hbm, pslot_hbm,
                   slot_v, prob_v, ids_v, ps_v, rows_v,
                   gs0, gs1, os0, os1):
    wid = lax.axis_index("s") * NC + lax.axis_index("c")
    base = wid * RPT

    pltpu.sync_copy(slot_hbm, slot_v)
    pltpu.sync_copy(prob_hbm, prob_v)

    zero16 = jnp.zeros((16,), jnp.int32)

    def init_body(c, _):
        ids_v[pl.ds(c * 16, 16)] = zero16
        return 0

    lax.fori_loop(0, T // 16, init_body, 0)

    i16 = lax.iota(jnp.int32, 16)

    def scat_body(c, _):
        sv = slot_v[pl.ds(c * 16, 16)]
        tok = i16 + c * 16
        m = sv < T
        plsc.store_scatter(ids_v, [sv], tok, mask=m)
        plsc.store_scatter(ps_v, [sv], prob_v[pl.ds(c * 16, 16)], mask=m)
        return 0

    lax.fori_loop(0, T // 16, scat_body, 0)

    pltpu.sync_copy(ps_v.at[pl.ds(base, RPT)], pslot_hbm.at[pl.ds(base, RPT)])

    _pipelined_gather(x_hbm, ids_v.at[pl.ds(base, RPT)], xe_hbm, base,
                      rows_v, (gs0, gs1), (os0, os1))


def _dispatch(x_flat, slot, prob):
    mesh = plsc.VectorSubcoreMesh(core_axis_name="c", subcore_axis_name="s")
    k = pl.kernel(
        _dispatch_body,
        compiler_params=pltpu.CompilerParams(needs_layout_passes=False),
        out_type=[
            jax.ShapeDtypeStruct((T, D), jnp.float32),
            jax.ShapeDtypeStruct((T,), jnp.float32),
        ],
        mesh=mesh,
        scratch_types=[
            pltpu.VMEM((T,), jnp.int32),
            pltpu.VMEM((T,), jnp.float32),
            pltpu.VMEM((T,), jnp.int32),
            pltpu.VMEM((T,), jnp.float32),
            pltpu.VMEM((2, GCH, D), jnp.float32),
            pltpu.SemaphoreType.DMA,
            pltpu.SemaphoreType.DMA,
            pltpu.SemaphoreType.DMA,
            pltpu.SemaphoreType.DMA,
        ],
    )
    return k(x_flat, slot, prob)


def _ffn_body(x_ref, w1_ref, w2_ref, p_ref, out_ref, acc_ref):
    e = pl.program_id(0)
    f = pl.program_id(1)

    @pl.when(e < E)
    def _():
        xb = x_ref[0].astype(jnp.bfloat16)
        w1 = w1_ref[0].astype(jnp.bfloat16)
        h = lax.dot_general(xb, w1, (((1,), (1,)), ((), ())),
                            preferred_element_type=jnp.float32)
        h = jnp.maximum(h, 0.0).astype(jnp.bfloat16)
        w2 = w2_ref[0].astype(jnp.bfloat16)
        part = lax.dot_general(h, w2, (((1,), (1,)), ((), ())),
                               preferred_element_type=jnp.float32)

        @pl.when(f == 0)
        def _():
            acc_ref[...] = part

        @pl.when(f > 0)
        def _():
            acc_ref[...] = acc_ref[...] + part

        @pl.when(f == NFFB - 1)
        def _():
            pv = p_ref[0, 0][:, None]
            out_ref[0] = acc_ref[...] * pv

    @pl.when(e == E)
    def _():
        out_ref[0] = jnp.zeros_like(out_ref[0])


def _ffn(xe, W1, W2, pslot):
    xe3 = xe.reshape(E, CAP, D)
    p3 = pslot.reshape(E, 1, CAP)
    clamp = lambda e: jnp.minimum(e, E - 1)
    out = pl.pallas_call(
        _ffn_body,
        grid=(E + 1, NFFB),
        in_specs=[
            pl.BlockSpec((1, CAP, D), lambda e, f: (clamp(e), 0, 0)),
            pl.BlockSpec((1, FFB, D), lambda e, f: (clamp(e), f, 0)),
            pl.BlockSpec((1, D, FFB), lambda e, f: (clamp(e), 0, f)),
            pl.BlockSpec((1, 1, CAP), lambda e, f: (clamp(e), 0, 0)),
        ],
        out_specs=pl.BlockSpec((1, CAP, D), lambda e, f: (e, 0, 0)),
        out_shape=jax.ShapeDtypeStruct((E + 1, CAP, D), jnp.float32),
        scratch_shapes=[pltpu.VMEM((CAP, D), jnp.float32)],
    )(xe3, W1, W2, p3)
    return out.reshape((E + 1) * CAP, D)


def _collect_body(res_hbm, slot_hbm, y_hbm, slot_v, rows_v,
                  gs0, gs1, os0, os1):
    wid = lax.axis_index("s") * NC + lax.axis_index("c")
    base = wid * RPT

    pltpu.sync_copy(slot_hbm.at[pl.ds(base, RPT)], slot_v)

    _pipelined_gather(res_hbm, slot_v, y_hbm, base,
                      rows_v, (gs0, gs1), (os0, os1))


def _collect(result, slot):
    mesh = plsc.VectorSubcoreMesh(core_axis_name="c", subcore_axis_name="s")
    k = pl.kernel(
        _collect_body,
        compiler_params=pltpu.CompilerParams(needs_layout_passes=False),
        out_type=jax.ShapeDtypeStruct((T, D), jnp.float32),
        mesh=mesh,
        scratch_types=[
            pltpu.VMEM((RPT,), jnp.int32),
            pltpu.VMEM((2, GCH, D), jnp.float32),
            pltpu.SemaphoreType.DMA,
            pltpu.SemaphoreType.DMA,
            pltpu.SemaphoreType.DMA,
            pltpu.SemaphoreType.DMA,
        ],
    )
    return k(result, slot)


@jax.jit
def kernel(x, Wr, W1, W2):
    x_flat = x.reshape(T, D)
    slot, prob = _router(x_flat, Wr)
    xe, pslot = _dispatch(x_flat, slot, prob)
    result = _ffn(xe, W1, W2, pslot)
    y = _collect(result, slot)
    return y.reshape(B, S, D)
